# baseline (speedup 1.0000x reference)
import functools

import jax
import jax.numpy as jnp
from jax import lax
from jax.experimental import pallas as pl
from jax.experimental.pallas import tpu as pltpu
from jax.experimental.pallas import tpu_sc as plsc

VOCAB = 1000000
SEQ = 200
DIM = 64
BATCH = 4096

NC = 2
NS = 16
NW = NC * NS
LANES = 16
PADD = 128

BPW = BATCH // NW
SPLIT = 128

FMT_CHUNK = 160
FMT_NCHUNK = VOCAB // FMT_CHUNK

_MESH = plsc.VectorSubcoreMesh(core_axis_name="c", subcore_axis_name="s")
_PARAMS = pltpu.CompilerParams(use_tc_tiling_on_sc=True)


@functools.partial(
    pl.kernel,
    mesh=_MESH,
    out_type=jax.ShapeDtypeStruct((VOCAB, PADD), jnp.float32),
    compiler_params=_PARAMS,
    scratch_types=[
        pltpu.VMEM((2, FMT_CHUNK, DIM), jnp.float32),
        pltpu.VMEM((2, FMT_CHUNK, PADD), jnp.float32),
        pltpu.SemaphoreType.DMA((2,)),
        pltpu.SemaphoreType.DMA((2,)),
    ],
)
def _fmt_kernel(table_hbm, table2_hbm, bin_v, bout_v, sem_i, sem_o):
    wid = lax.axis_index("s") * NC + lax.axis_index("c")
    nmine = (FMT_NCHUNK - wid + NW - 1) // NW

    def in_copy(t, slot):
        c = t * NW + wid
        return pltpu.make_async_copy(
            table_hbm.at[pl.ds(c * FMT_CHUNK, FMT_CHUNK)],
            bin_v.at[slot], sem_i.at[slot])

    def out_copy(t, slot):
        c = t * NW + wid
        return pltpu.make_async_copy(
            bout_v.at[slot],
            table2_hbm.at[pl.ds(c * FMT_CHUNK, FMT_CHUNK)], sem_o.at[slot])

    @pl.when(nmine > 0)
    def _():
        in_copy(0, 0).start()

    def fmt_body(t, carry):
        slot = lax.rem(t, 2)

        @pl.when(t + 1 < nmine)
        def _():
            in_copy(t + 1, 1 - slot).start()

        in_copy(t, slot).wait()

        @pl.when(t >= 2)
        def _():
            out_copy(t - 2, slot).wait()

        def repack(j, inner):
            for v in range(DIM // LANES):
                vec = bin_v[slot, j, pl.ds(v * LANES, LANES)]
                bout_v[slot, j, pl.ds(v * LANES, LANES)] = vec
            return inner

        lax.fori_loop(0, FMT_CHUNK, repack, 0, unroll=4)
        out_copy(t, slot).start()
        return carry

    lax.fori_loop(0, nmine, fmt_body, 0)

    @pl.when(nmine > 0)
    def _():
        out_copy(nmine - 1, lax.rem(nmine - 1, 2)).wait()

    @pl.when(nmine > 1)
    def _():
        out_copy(nmine - 2, lax.rem(nmine - 2, 2)).wait()


@functools.partial(
    pl.kernel,
    mesh=_MESH,
    out_type=jax.ShapeDtypeStruct((BATCH, SEQ, DIM), jnp.float32),
    compiler_params=_PARAMS,
    scratch_types=[
        pltpu.VMEM((2, SEQ), jnp.int32),
        pltpu.VMEM((SEQ // 2, PADD), jnp.float32),
        pltpu.VMEM((2, SEQ, PADD), jnp.float32),
        pltpu.VMEM((2, SEQ, DIM), jnp.float32),
        pltpu.SemaphoreType.DMA((2,)),
        pltpu.SemaphoreType.DMA((2,)),
        pltpu.SemaphoreType.DMA((2,)),
    ],
)
def _emb_kernel(idx_hbm, pos_hbm, table2_hbm, out_hbm,
                idx_v, posp_v, rows_v, stage_v, sem_x, sem_g, sem_s):
    wid = lax.axis_index("s") * NC + lax.axis_index("c")
    row0 = wid * BPW

    pltpu.sync_copy(pos_hbm, stage_v.at[0])

    def pack_pos(i, inner):
        for half in range(2):
            for v in range(DIM // LANES):
                vec = stage_v[0, i * 2 + half, pl.ds(v * LANES, LANES)]
                posp_v[i, pl.ds(half * DIM + v * LANES, LANES)] = vec
        return inner

    lax.fori_loop(0, SEQ // 2, pack_pos, 0, unroll=2)

    def idx_copy(r, slot):
        return pltpu.make_async_copy(
            idx_hbm.at[row0 + r], idx_v.at[slot], sem_x.at[slot])

    def gather_copies(slot):
        return (
            pltpu.make_async_copy(
                table2_hbm.at[idx_v.at[slot, pl.ds(0, SPLIT)]],
                rows_v.at[slot, pl.ds(0, SPLIT)], sem_g.at[slot]),
            pltpu.make_async_copy(
                table2_hbm.at[idx_v.at[slot, pl.ds(SPLIT, SEQ - SPLIT)]],
                rows_v.at[slot, pl.ds(SPLIT, SEQ - SPLIT)], sem_g.at[slot]),
        )

    def store_copy(r, slot):
        return pltpu.make_async_copy(
            stage_v.at[slot], out_hbm.at[row0 + r], sem_s.at[slot])

    idx_copy(0, 0).start()
    idx_copy(1, 1).start()
    idx_copy(0, 0).wait()
    for c in gather_copies(0):
        c.start()

    def row_body(i, carry):
        for b in range(2):
            r = i * 2 + b

            @pl.when(r + 1 < BPW)
            def _():
                idx_copy(r + 1, 1 - b).wait()
                for c in gather_copies(1 - b):
                    c.start()

            for c in gather_copies(b):
                c.wait()

            @pl.when(r + 2 < BPW)
            def _():
                idx_copy(r + 2, b).start()

            @pl.when(r >= 2)
            def _():
                store_copy(r - 2, b).wait()

            def add_pos(i2, inner):
                for half in range(2):
                    for v in range(DIM // LANES):
                        vec = rows_v[b, i2 * 2 + half, pl.ds(v * LANES, LANES)]
                        pvec = posp_v[i2, pl.ds(half * DIM + v * LANES, LANES)]
                        stage_v[b, i2 * 2 + half, pl.ds(v * LANES, LANES)] = (
                            vec + pvec)
                return inner

            lax.fori_loop(0, SEQ // 2, add_pos, 0, unroll=2)
            store_copy(r, b).start()
        return carry

    lax.fori_loop(0, BPW // 2, row_body, 0)

    store_copy(BPW - 2, 0).wait()
    store_copy(BPW - 1, 1).wait()


def kernel(inputs, word_table, pos_table):
    table2 = _fmt_kernel(word_table)
    return _emb_kernel(inputs.astype(jnp.int32), pos_table, table2)
